# Initial kernel scaffold; baseline (speedup 1.0000x reference)
import jax
import jax.numpy as jnp
from jax.experimental import pallas as pl


def kernel(inputs, weight):
    raise NotImplementedError("write your pallas kernel here")


# speedup vs baseline: 2.5531x; 2.5531x over previous
import functools

import jax
import jax.numpy as jnp
from jax import lax
from jax.experimental import pallas as pl
from jax.experimental.pallas import tpu as pltpu
from jax.experimental.pallas import tpu_sc as plsc

EMBED = 64
BATCH = 16384
LABELS = 50

NC, NS = 2, 16
NW = NC * NS
BB = 128
NB = BATCH // BB
BPW = NB // NW
LANES = 16
NVR = EMBED // LANES


def _sc_embed_sum(weight, idx_r):
    mesh = plsc.VectorSubcoreMesh(core_axis_name="c", subcore_axis_name="s")

    @functools.partial(
        pl.kernel,
        out_type=jax.ShapeDtypeStruct((BATCH, EMBED), jnp.float32),
        mesh=mesh,
        compiler_params=pltpu.CompilerParams(use_tc_tiling_on_sc=False),
        scratch_types=[
            pltpu.VMEM((LABELS, BB), jnp.int32),
            pltpu.VMEM((BB, EMBED), jnp.float32),
            pltpu.VMEM((BB, EMBED), jnp.float32),
            pltpu.VMEM((BB, EMBED), jnp.float32),
            pltpu.SemaphoreType.DMA,
            pltpu.SemaphoreType.DMA,
        ],
    )
    def k(w_hbm, idx_hbm, out_hbm, idx_v, acc, buf0, buf1, sem0, sem1):
        wid = lax.axis_index("s") * NC + lax.axis_index("c")
        bufs = (buf0, buf1)
        sems = (sem0, sem1)
        zero = jnp.zeros((LANES,), jnp.float32)

        def gather_start(l, b):
            pltpu.make_async_copy(w_hbm.at[idx_v.at[l]], bufs[b], sems[b]).start()

        def gather_wait(b):
            pltpu.make_async_copy(w_hbm.at[idx_v.at[0]], bufs[b], sems[b]).wait()

        def accum(b):
            buf = bufs[b]

            def rbody(i, _):
                r0 = i * 4
                for kk in range(4):
                    for c in range(NVR):
                        sl = pl.ds(c * LANES, LANES)
                        plsc.addupdate(acc.at[r0 + kk, sl], buf[r0 + kk, sl])
                return 0

            lax.fori_loop(0, BB // 4, rbody, 0)

        def block_body(kblk, _):
            jb = wid * BPW + kblk
            pltpu.sync_copy(idx_hbm.at[jb], idx_v)

            def zbody(i, _):
                r0 = i * 4
                for kk in range(4):
                    for c in range(NVR):
                        acc[r0 + kk, pl.ds(c * LANES, LANES)] = zero
                return 0

            lax.fori_loop(0, BB // 4, zbody, 0)

            gather_start(0, 0)
            gather_start(1, 1)

            def pair_body(i, _):
                for b in range(2):
                    l = 2 * i + b
                    gather_wait(b)
                    accum(b)
                    gather_start(l + 2, b)
                return 0

            lax.fori_loop(0, LABELS // 2 - 1, pair_body, 0)

            for b in range(2):
                gather_wait(b)
                accum(b)

            pltpu.sync_copy(acc, out_hbm.at[pl.ds(jb * BB, BB)])
            return 0

        lax.fori_loop(0, BPW, block_body, 0)

    return k(weight, idx_r)


def kernel(inputs, weight):
    idx = inputs.astype(jnp.int32)
    idx_r = idx.reshape(NB, BB, LABELS).transpose(0, 2, 1)
    return _sc_embed_sum(weight, idx_r)
